# Initial kernel scaffold; baseline (speedup 1.0000x reference)
import jax
import jax.numpy as jnp
from jax.experimental import pallas as pl


def kernel(edge_index, batch, W_enc, b_enc, W1, b1, W2, b2, w_pool, Wd1, bd1, Wd2, bd2):
    raise NotImplementedError("write your pallas kernel here")


# speedup vs baseline: 10.5998x; 10.5998x over previous
import functools

import jax
import jax.numpy as jnp
from jax import lax
from jax.experimental import pallas as pl
from jax.experimental.pallas import tpu as pltpu
from jax.experimental.pallas import tpu_sc as plsc

N = 10000
E = 320000
H = 128
OUT = 10
G = 64
NUM_BLOCKS = 3
NUM_LAYERS = 4
RATIO = 0.5

NP = 10240
RB = NP // 128
NTILES = 32
HALF_N = NP // 2
NCHUNK = 84
DCHUNK = 80
CH = 128
EHALF = 16 * NCHUNK * CH
RPS = NP // 16
RPS2 = HALF_N // 16

_INT_MIN = -2147483648
_LO0 = -1065353218
_HI0 = 1065353217


def _sc_mesh():
    return plsc.VectorSubcoreMesh(core_axis_name="c", subcore_axis_name="s")


@functools.lru_cache(maxsize=None)
def _build_sc_agg():
  @functools.partial(
    pl.kernel,
    mesh=_sc_mesh(),
    out_type=jax.ShapeDtypeStruct((NP, H), jnp.float32),
    scratch_types=[
        pltpu.VMEM((NCHUNK, CH), jnp.int32),
        pltpu.VMEM((NCHUNK, CH), jnp.int32),
        pltpu.VMEM((CH, H), jnp.float32),
        pltpu.VMEM((CH, H), jnp.float32),
        pltpu.VMEM_SHARED((HALF_N, H), jnp.float32),
        pltpu.SemaphoreType.DMA,
        pltpu.SemaphoreType.DMA,
    ],
)
  def sc_agg(x_hbm, srcr_hbm, dstr_hbm, zeros_hbm, out_hbm,
               src_v, dst_v, rows0_v, rows1_v, acc, sem0, sem1):
      cc = lax.axis_index("c")
      ss = lax.axis_index("s")
      wid = cc * 16 + ss
      pltpu.sync_copy(srcr_hbm.at[wid], src_v)
      pltpu.sync_copy(dstr_hbm.at[wid], dst_v)

      def zstep(t, carry):
          pltpu.sync_copy(zeros_hbm, acc.at[pl.ds(ss * RPS2 + t * 64, 64)])
          return carry

      lax.fori_loop(0, RPS2 // 64, zstep, 0)
      plsc.subcore_barrier()

      pltpu.make_async_copy(x_hbm.at[src_v.at[0]], rows0_v, sem0).start()
      pltpu.make_async_copy(x_hbm.at[src_v.at[1]], rows1_v, sem1).start()

      def step(p, carry):
          for b in range(2):
              j = p * 2 + b
              buf = rows0_v if b == 0 else rows1_v
              sem = sem0 if b == 0 else sem1
              pltpu.make_async_copy(x_hbm.at[src_v.at[j]], buf, sem).wait()
              pltpu.sync_copy(buf, acc.at[dst_v.at[j]], add=True)

              @pl.when(j + 2 < NCHUNK)
              def _():
                  pltpu.make_async_copy(x_hbm.at[src_v.at[j + 2]], buf, sem).start()
          return carry

      lax.fori_loop(0, NCHUNK // 2, step, 0)
      plsc.subcore_barrier()
      pltpu.sync_copy(acc.at[pl.ds(ss * RPS2, RPS2)],
                      out_hbm.at[pl.ds(cc * HALF_N + ss * RPS2, RPS2)])

  return sc_agg


@functools.lru_cache(maxsize=None)
def _build_sc_deg():
  @functools.partial(
    pl.kernel,
    mesh=_sc_mesh(),
    out_type=jax.ShapeDtypeStruct((2 * NP,), jnp.float32),
    scratch_types=[
        pltpu.VMEM((DCHUNK, CH), jnp.int32),
        pltpu.VMEM((CH,), jnp.float32),
        pltpu.VMEM_SHARED((NP,), jnp.float32),
    ],
)
  def sc_deg(srcr_hbm, ones_hbm, zeros_hbm, out_hbm, src_v, ones_v, acc):
      cc = lax.axis_index("c")
      ss = lax.axis_index("s")
      wid = cc * 16 + ss
      pltpu.sync_copy(srcr_hbm.at[wid], src_v)
      pltpu.sync_copy(ones_hbm, ones_v)
      pltpu.sync_copy(zeros_hbm, acc.at[pl.ds(ss * RPS, RPS)])
      plsc.subcore_barrier()

      def step(j, carry):
          pltpu.sync_copy(ones_v, acc.at[src_v.at[j]], add=True)
          return carry

      lax.fori_loop(0, DCHUNK, step, 0)
      plsc.subcore_barrier()
      pltpu.sync_copy(acc.at[pl.ds(ss * RPS, RPS)],
                      out_hbm.at[pl.ds(cc * NP + ss * RPS, RPS)])

  return sc_deg


def _sc_agg(x, srcr2, dstr2, zeros_rows):
    return _build_sc_agg()(x, srcr2, dstr2, zeros_rows)


def _sc_deg(srcrd, ones_row, zeros_deg):
    return _build_sc_deg()(srcrd, ones_row, zeros_deg)


def _enc_body(deg_ref, wenc_ref, benc_ref, pad_ref, x_ref):
    degc = deg_ref[0] + deg_ref[1]
    w = wenc_ref[...].reshape(1, 1, H)
    b = benc_ref[...].reshape(1, 1, H)
    x3 = jnp.maximum(degc[:, :, None] * w + b, 0.0)
    x3 = x3 * pad_ref[...][:, :, None]
    x_ref[...] = x3.reshape(NP, H)


def _tc_enc(deg2, W_enc, b_enc, padmask):
    return pl.pallas_call(
        _enc_body,
        out_shape=jax.ShapeDtypeStruct((NP, H), jnp.float32),
    )(deg2, W_enc, b_enc, padmask)


def _layer_body(x_ref, agg_ref, w1_ref, b1_ref, w2_ref, b2_ref, valid_ref, o_ref):
    h = x_ref[...] + agg_ref[...]
    h = jnp.maximum(jnp.dot(h, w1_ref[...], preferred_element_type=jnp.float32)
                    + b1_ref[...][None, :], 0.0)
    y = jnp.maximum(jnp.dot(h, w2_ref[...], preferred_element_type=jnp.float32)
                    + b2_ref[...][None, :], 0.0)
    y3 = y.reshape(RB, 128, H) * valid_ref[...][:, :, None]
    o_ref[...] = y3.reshape(NP, H)


def _tc_layer(x, agg, W1l, b1l, W2l, b2l, valid):
    return pl.pallas_call(
        _layer_body,
        out_shape=jax.ShapeDtypeStruct((NP, H), jnp.float32),
    )(x, agg, W1l, b1l, W2l, b2l, valid)


def _hp32(a, b):
    return lax.dot_general(a, b, (((1,), (0,)), ((), ())),
                           precision=lax.Precision.HIGHEST,
                           preferred_element_type=jnp.float32)


def _pool_body(x_ref, w_ref, valid_ref, ohng_ref, ohgn_ref, ut_ref, slt_ref,
               sut_ref, xo_ref, vo_ref):
    x = x_ref[...]
    w = w_ref[0, 0]
    valid = valid_ref[...]
    ohng = ohng_ref[...]
    ohgn = ohgn_ref[...]

    x3 = x.reshape(RB, 128, H)
    s = jnp.sum(x3 * w, axis=-1)
    score = jnp.tanh(s / jnp.sqrt(w * w))

    sb = lax.bitcast_convert_type(score, jnp.int32)
    u = jnp.where(sb >= 0, sb, sb ^ jnp.int32(0x7FFFFFFF))
    u = jnp.where(valid > 0.5, u, _INT_MIN)

    n_valid = _hp32(valid.reshape(1, NP), ohng)
    kf = jnp.floor((n_valid + 1.0) * 0.5)

    def bs(i, c):
        lo, hi = c
        mid = lax.shift_right_arithmetic(lo + hi + 1, 1)
        ind = jnp.where(u >= mid, 1.0, 0.0)
        cnt = _hp32(ind.reshape(1, NP), ohng)
        okf = jnp.where(cnt >= kf, 1.0, 0.0)
        okn = _hp32(okf, ohgn).reshape(RB, 128)
        ok = okn > 0.5
        return jnp.where(ok, mid, lo), jnp.where(ok, hi, mid - 1)

    lo0 = jnp.full((RB, 128), _LO0, jnp.int32)
    hi0 = jnp.full((RB, 128), _HI0, jnp.int32)
    tstar, _ = lax.fori_loop(0, 31, bs, (lo0, hi0))

    gt = jnp.where(u > tstar, 1.0, 0.0)
    tied = jnp.where(u == tstar, 1.0, 0.0)
    cnt_gt = _hp32(gt.reshape(1, NP), ohng)
    rem = kf - cnt_gt
    rem_n = _hp32(rem, ohgn).reshape(RB, 128)

    prefix = _hp32(tied, ut_ref[...])
    rowtot = jnp.sum(tied, axis=-1, keepdims=True) * jnp.ones((1, 128), jnp.float32)
    offs = _hp32(slt_ref[...], rowtot)
    c_incl = prefix + offs
    t_g = _hp32(tied.reshape(1, NP), ohng)
    cb = _hp32(t_g, sut_ref[...])
    cb_n = _hp32(cb, ohgn).reshape(RB, 128)
    tie_rank = c_incl - tied - cb_n

    keep = gt + tied * jnp.where(tie_rank < rem_n, 1.0, 0.0)
    xo3 = x3 * (score * keep)[:, :, None]
    xo_ref[...] = xo3.reshape(NP, H)
    vo_ref[...] = keep


def _tc_pool(x, w_blk, valid, ohng, ohgn, ut, slt, sut):
    return pl.pallas_call(
        _pool_body,
        out_shape=[jax.ShapeDtypeStruct((NP, H), jnp.float32),
                   jax.ShapeDtypeStruct((RB, 128), jnp.float32)],
    )(x, w_blk, valid, ohng, ohgn, ut, slt, sut)


def _final_body(x_ref, ohgn_ref, wd1_ref, bd1_ref, wd2_ref, bd2_ref, o_ref):
    pooled = jnp.dot(ohgn_ref[...], x_ref[...], preferred_element_type=jnp.float32)
    h = jnp.maximum(jnp.dot(pooled, wd1_ref[...], preferred_element_type=jnp.float32)
                    + bd1_ref[...][None, :], 0.0)
    o_ref[...] = (jnp.dot(h, wd2_ref[...], preferred_element_type=jnp.float32)
                  + bd2_ref[...][None, :])


def _tc_final(x, ohgn, Wd1, bd1, Wd2, bd2):
    return pl.pallas_call(
        _final_body,
        out_shape=jax.ShapeDtypeStruct((G, OUT), jnp.float32),
    )(x, ohgn, Wd1, bd1, Wd2, bd2)


def kernel(edge_index, batch, W_enc, b_enc, W1, b1, W2, b2, w_pool, Wd1, bd1, Wd2, bd2):
    src = edge_index[0].astype(jnp.int32)
    dst = edge_index[1].astype(jnp.int32)

    npad = DCHUNK * CH - E // NTILES
    pad_src = (N + (jnp.arange(npad, dtype=jnp.int32) % 240)).astype(jnp.int32)
    srcr = jnp.concatenate(
        [src.reshape(NTILES, E // NTILES),
         jnp.broadcast_to(pad_src, (NTILES, npad))], axis=1)
    srcrd = srcr.reshape(NTILES, DCHUNK, CH)

    flag = (dst >= HALF_N).astype(jnp.int32)
    perm = jnp.argsort(flag, stable=True)
    srcp = src[perm]
    dstp = dst[perm]
    count0 = jnp.int32(E) - jnp.sum(flag)
    idx = jnp.arange(EHALF, dtype=jnp.int32)
    pad_s = N + (idx % 240)
    pad_d = (idx % 128) * (HALF_N // 128)
    s0 = jnp.where(idx < count0, srcp[:EHALF], pad_s)
    d0 = jnp.where(idx < count0, dstp[:EHALF], pad_d)
    idx1 = idx + count0
    gi = jnp.minimum(idx1, E - 1)
    s1 = jnp.where(idx1 < E, jnp.take(srcp, gi), pad_s)
    d1 = jnp.where(idx1 < E, jnp.take(dstp, gi) - HALF_N, pad_d)
    srcr2 = jnp.stack([s0, s1]).reshape(NTILES, NCHUNK, CH)
    dstr2 = jnp.stack([d0, d1]).reshape(NTILES, NCHUNK, CH)

    zeros_rows = jnp.zeros((64, H), jnp.float32)
    zeros_deg = jnp.zeros((RPS,), jnp.float32)
    ones_row = jnp.ones((CH,), jnp.float32)

    batch_pad = jnp.concatenate(
        [batch.astype(jnp.int32), jnp.full((NP - N,), G, jnp.int32)])
    ohng = (batch_pad[:, None] == jnp.arange(G, dtype=jnp.int32)[None, :]
            ).astype(jnp.float32)
    ohgn = ohng.T
    padmask = (jnp.arange(NP, dtype=jnp.int32) < N).astype(jnp.float32).reshape(RB, 128)
    ii = jnp.arange(128, dtype=jnp.int32)
    ut = (ii[:, None] <= ii[None, :]).astype(jnp.float32)
    rr = jnp.arange(RB, dtype=jnp.int32)
    slt = (rr[None, :] < rr[:, None]).astype(jnp.float32)
    gg = jnp.arange(G, dtype=jnp.int32)
    sut = (gg[:, None] < gg[None, :]).astype(jnp.float32)

    deg2 = _sc_deg(srcrd, ones_row, zeros_deg).reshape(2, RB, 128)
    x = _tc_enc(deg2, W_enc, b_enc, padmask)
    valid = padmask
    layer = 0
    for blk in range(NUM_BLOCKS):
        for _ in range(NUM_LAYERS):
            agg = _sc_agg(x, srcr2, dstr2, zeros_rows)
            x = _tc_layer(x, agg, W1[layer], b1[layer], W2[layer], b2[layer], valid)
            layer += 1
        if blk < NUM_BLOCKS - 1:
            w_blk = w_pool[blk].reshape(1, 1)
            x, valid = _tc_pool(x, w_blk, valid, ohng, ohgn, ut, slt, sut)
    return _tc_final(x, ohgn, Wd1, bd1, Wd2, bd2)


# speedup vs baseline: 13.3871x; 1.2630x over previous
import functools

import jax
import jax.numpy as jnp
from jax import lax
from jax.experimental import pallas as pl
from jax.experimental.pallas import tpu as pltpu
from jax.experimental.pallas import tpu_sc as plsc

N = 10000
E = 320000
H = 128
OUT = 10
G = 64
NUM_BLOCKS = 3
NUM_LAYERS = 4
RATIO = 0.5

NP = 10240
RB = NP // 128
NTILES = 32
HALF_N = NP // 2
NCHUNK = 84
DCHUNK = 80
CH = 128
EHALF = 16 * NCHUNK * CH
RPS = NP // 16
RPS2 = HALF_N // 16

_INT_MIN = -2147483648
_LO0 = -1065353218
_HI0 = 1065353217


def _sc_mesh():
    return plsc.VectorSubcoreMesh(core_axis_name="c", subcore_axis_name="s")


@functools.lru_cache(maxsize=None)
def _build_sc_agg():
  @functools.partial(
    pl.kernel,
    mesh=_sc_mesh(),
    out_type=jax.ShapeDtypeStruct((NP, H), jnp.float32),
    scratch_types=[
        pltpu.VMEM((NCHUNK, CH), jnp.int32),
        pltpu.VMEM((NCHUNK, CH), jnp.int32),
        pltpu.VMEM((CH, H), jnp.float32),
        pltpu.VMEM((CH, H), jnp.float32),
        pltpu.VMEM_SHARED((HALF_N, H), jnp.float32),
        pltpu.SemaphoreType.DMA,
        pltpu.SemaphoreType.DMA,
    ],
)
  def sc_agg(x_hbm, srcr_hbm, dstr_hbm, zeros_hbm, out_hbm,
               src_v, dst_v, rows0_v, rows1_v, acc, sem0, sem1):
      cc = lax.axis_index("c")
      ss = lax.axis_index("s")
      wid = cc * 16 + ss
      pltpu.sync_copy(srcr_hbm.at[wid], src_v)
      pltpu.sync_copy(dstr_hbm.at[wid], dst_v)

      def zstep(t, carry):
          pltpu.sync_copy(zeros_hbm, acc.at[pl.ds(ss * RPS2 + t * 64, 64)])
          return carry

      lax.fori_loop(0, RPS2 // 64, zstep, 0)
      plsc.subcore_barrier()

      pltpu.make_async_copy(x_hbm.at[src_v.at[0]], rows0_v, sem0).start()
      pltpu.make_async_copy(x_hbm.at[src_v.at[1]], rows1_v, sem1).start()

      def step(p, carry):
          for b in range(2):
              j = p * 2 + b
              buf = rows0_v if b == 0 else rows1_v
              sem = sem0 if b == 0 else sem1
              pltpu.make_async_copy(x_hbm.at[src_v.at[j]], buf, sem).wait()
              pltpu.sync_copy(buf, acc.at[dst_v.at[j]], add=True)

              @pl.when(j + 2 < NCHUNK)
              def _():
                  pltpu.make_async_copy(x_hbm.at[src_v.at[j + 2]], buf, sem).start()
          return carry

      lax.fori_loop(0, NCHUNK // 2, step, 0)
      plsc.subcore_barrier()
      pltpu.sync_copy(acc.at[pl.ds(ss * RPS2, RPS2)],
                      out_hbm.at[pl.ds(cc * HALF_N + ss * RPS2, RPS2)])

  return sc_agg


@functools.lru_cache(maxsize=None)
def _build_sc_deg():
  @functools.partial(
    pl.kernel,
    mesh=_sc_mesh(),
    out_type=jax.ShapeDtypeStruct((2 * NP,), jnp.float32),
    scratch_types=[
        pltpu.VMEM((DCHUNK, CH), jnp.int32),
        pltpu.VMEM((CH,), jnp.float32),
        pltpu.VMEM_SHARED((NP,), jnp.float32),
    ],
)
  def sc_deg(srcr_hbm, ones_hbm, zeros_hbm, out_hbm, src_v, ones_v, acc):
      cc = lax.axis_index("c")
      ss = lax.axis_index("s")
      wid = cc * 16 + ss
      pltpu.sync_copy(srcr_hbm.at[wid], src_v)
      pltpu.sync_copy(ones_hbm, ones_v)
      pltpu.sync_copy(zeros_hbm, acc.at[pl.ds(ss * RPS, RPS)])
      plsc.subcore_barrier()

      def step(j, carry):
          pltpu.sync_copy(ones_v, acc.at[src_v.at[j]], add=True)
          return carry

      lax.fori_loop(0, DCHUNK, step, 0)
      plsc.subcore_barrier()
      pltpu.sync_copy(acc.at[pl.ds(ss * RPS, RPS)],
                      out_hbm.at[pl.ds(cc * NP + ss * RPS, RPS)])

  return sc_deg


EDGE_T = NTILES * NCHUNK * CH
EPS = EDGE_T // 16


@functools.lru_cache(maxsize=None)
def _build_sc_part():
  @functools.partial(
    pl.kernel,
    mesh=_sc_mesh(),
    out_type=jax.ShapeDtypeStruct((2 * EDGE_T,), jnp.int32),
    scratch_types=[
        pltpu.VMEM((NCHUNK, CH), jnp.int32),
        pltpu.VMEM((NCHUNK, CH), jnp.int32),
        pltpu.VMEM_SHARED((EDGE_T,), jnp.int32),
    ],
)
  def sc_part(pos_hbm, vals_hbm, zeros_hbm, out_hbm, pos_v, val_v, buf):
      cc = lax.axis_index("c")
      ss = lax.axis_index("s")
      wid = cc * 16 + ss
      pltpu.sync_copy(pos_hbm.at[wid], pos_v)
      pltpu.sync_copy(vals_hbm.at[wid], val_v)

      def zstep(t, carry):
          pltpu.sync_copy(zeros_hbm, buf.at[pl.ds(ss * EPS + t * 2688, 2688)])
          return carry

      lax.fori_loop(0, EPS // 2688, zstep, 0)
      plsc.subcore_barrier()

      def step(j, carry):
          pltpu.sync_copy(val_v.at[j], buf.at[pos_v.at[j]])
          return carry

      lax.fori_loop(0, NCHUNK, step, 0)
      plsc.subcore_barrier()
      pltpu.sync_copy(buf.at[pl.ds(ss * EPS, EPS)],
                      out_hbm.at[pl.ds(cc * EDGE_T + ss * EPS, EPS)])

  return sc_part


def _sc_agg(x, srcr2, dstr2, zeros_rows):
    return _build_sc_agg()(x, srcr2, dstr2, zeros_rows)


def _sc_part(pos, packed, zeros_i32):
    return _build_sc_part()(pos, packed, zeros_i32)


def _sc_deg(srcrd, ones_row, zeros_deg):
    return _build_sc_deg()(srcrd, ones_row, zeros_deg)


def _enc_body(deg_ref, wenc_ref, benc_ref, pad_ref, x_ref):
    degc = deg_ref[0] + deg_ref[1]
    w = wenc_ref[...].reshape(1, 1, H)
    b = benc_ref[...].reshape(1, 1, H)
    x3 = jnp.maximum(degc[:, :, None] * w + b, 0.0)
    x3 = x3 * pad_ref[...][:, :, None]
    x_ref[...] = x3.reshape(NP, H)


def _tc_enc(deg2, W_enc, b_enc, padmask):
    return pl.pallas_call(
        _enc_body,
        out_shape=jax.ShapeDtypeStruct((NP, H), jnp.float32),
    )(deg2, W_enc, b_enc, padmask)


def _layer_body(x_ref, agg_ref, w1_ref, b1_ref, w2_ref, b2_ref, valid_ref, o_ref):
    h = x_ref[...] + agg_ref[...]
    h = jnp.maximum(jnp.dot(h, w1_ref[...], preferred_element_type=jnp.float32)
                    + b1_ref[...][None, :], 0.0)
    y = jnp.maximum(jnp.dot(h, w2_ref[...], preferred_element_type=jnp.float32)
                    + b2_ref[...][None, :], 0.0)
    y3 = y.reshape(RB, 128, H) * valid_ref[...][:, :, None]
    o_ref[...] = y3.reshape(NP, H)


def _tc_layer(x, agg, W1l, b1l, W2l, b2l, valid):
    return pl.pallas_call(
        _layer_body,
        out_shape=jax.ShapeDtypeStruct((NP, H), jnp.float32),
    )(x, agg, W1l, b1l, W2l, b2l, valid)


def _hp32(a, b):
    return lax.dot_general(a, b, (((1,), (0,)), ((), ())),
                           precision=lax.Precision.HIGHEST,
                           preferred_element_type=jnp.float32)


def _pool_body(x_ref, w_ref, valid_ref, ohng_ref, ohgn_ref, ut_ref, slt_ref,
               sut_ref, xo_ref, vo_ref):
    x = x_ref[...]
    w = w_ref[0, 0]
    valid = valid_ref[...]
    ohng = ohng_ref[...]
    ohgn = ohgn_ref[...]

    x3 = x.reshape(RB, 128, H)
    s = jnp.sum(x3 * w, axis=-1)
    score = jnp.tanh(s / jnp.sqrt(w * w))

    sb = lax.bitcast_convert_type(score, jnp.int32)
    u = jnp.where(sb >= 0, sb, sb ^ jnp.int32(0x7FFFFFFF))
    u = jnp.where(valid > 0.5, u, _INT_MIN)

    n_valid = _hp32(valid.reshape(1, NP), ohng)
    kf = jnp.floor((n_valid + 1.0) * 0.5)

    def bs(i, c):
        lo, hi = c
        mid = lax.shift_right_arithmetic(lo + hi + 1, 1)
        ind = jnp.where(u >= mid, 1.0, 0.0)
        cnt = _hp32(ind.reshape(1, NP), ohng)
        okf = jnp.where(cnt >= kf, 1.0, 0.0)
        okn = _hp32(okf, ohgn).reshape(RB, 128)
        ok = okn > 0.5
        return jnp.where(ok, mid, lo), jnp.where(ok, hi, mid - 1)

    lo0 = jnp.full((RB, 128), _LO0, jnp.int32)
    hi0 = jnp.full((RB, 128), _HI0, jnp.int32)
    tstar, _ = lax.fori_loop(0, 31, bs, (lo0, hi0))

    gt = jnp.where(u > tstar, 1.0, 0.0)
    tied = jnp.where(u == tstar, 1.0, 0.0)
    cnt_gt = _hp32(gt.reshape(1, NP), ohng)
    rem = kf - cnt_gt
    rem_n = _hp32(rem, ohgn).reshape(RB, 128)

    prefix = _hp32(tied, ut_ref[...])
    rowtot = jnp.sum(tied, axis=-1, keepdims=True) * jnp.ones((1, 128), jnp.float32)
    offs = _hp32(slt_ref[...], rowtot)
    c_incl = prefix + offs
    t_g = _hp32(tied.reshape(1, NP), ohng)
    cb = _hp32(t_g, sut_ref[...])
    cb_n = _hp32(cb, ohgn).reshape(RB, 128)
    tie_rank = c_incl - tied - cb_n

    keep = gt + tied * jnp.where(tie_rank < rem_n, 1.0, 0.0)
    xo3 = x3 * (score * keep)[:, :, None]
    xo_ref[...] = xo3.reshape(NP, H)
    vo_ref[...] = keep


def _tc_pool(x, w_blk, valid, ohng, ohgn, ut, slt, sut):
    return pl.pallas_call(
        _pool_body,
        out_shape=[jax.ShapeDtypeStruct((NP, H), jnp.float32),
                   jax.ShapeDtypeStruct((RB, 128), jnp.float32)],
    )(x, w_blk, valid, ohng, ohgn, ut, slt, sut)


def _merge_body(a_ref, b_ref, s_ref, d_ref):
    m = a_ref[...] + b_ref[...]
    s_ref[...] = m & 0xFFFF
    d_ref[...] = lax.shift_right_arithmetic(m, 16)


def _tc_merge(a, b):
    return pl.pallas_call(
        _merge_body,
        out_shape=[jax.ShapeDtypeStruct((EDGE_T // 128, 128), jnp.int32),
                   jax.ShapeDtypeStruct((EDGE_T // 128, 128), jnp.int32)],
    )(a, b)


def _final_body(x_ref, ohgn_ref, wd1_ref, bd1_ref, wd2_ref, bd2_ref, o_ref):
    pooled = jnp.dot(ohgn_ref[...], x_ref[...], preferred_element_type=jnp.float32)
    h = jnp.maximum(jnp.dot(pooled, wd1_ref[...], preferred_element_type=jnp.float32)
                    + bd1_ref[...][None, :], 0.0)
    o_ref[...] = (jnp.dot(h, wd2_ref[...], preferred_element_type=jnp.float32)
                  + bd2_ref[...][None, :])


def _tc_final(x, ohgn, Wd1, bd1, Wd2, bd2):
    return pl.pallas_call(
        _final_body,
        out_shape=jax.ShapeDtypeStruct((G, OUT), jnp.float32),
    )(x, ohgn, Wd1, bd1, Wd2, bd2)


def kernel(edge_index, batch, W_enc, b_enc, W1, b1, W2, b2, w_pool, Wd1, bd1, Wd2, bd2):
    src = edge_index[0].astype(jnp.int32)
    dst = edge_index[1].astype(jnp.int32)

    npad = DCHUNK * CH - E // NTILES
    pad_src = (N + (jnp.arange(npad, dtype=jnp.int32) % 240)).astype(jnp.int32)
    srcr = jnp.concatenate(
        [src.reshape(NTILES, E // NTILES),
         jnp.broadcast_to(pad_src, (NTILES, npad))], axis=1)
    srcrd = srcr.reshape(NTILES, DCHUNK, CH)

    flag = (dst >= HALF_N).astype(jnp.int32)
    csum = jnp.cumsum(flag)
    cexc = csum - flag
    count0 = E - csum[-1]
    ar = jnp.arange(E, dtype=jnp.int32)
    pos_e = jnp.where(flag > 0, EHALF + cexc, ar - cexc)
    packed_e = src | ((dst - flag * HALF_N) << 16)
    ip = jnp.arange(2 * EHALF - E, dtype=jnp.int32)
    pos_p = jnp.where(ip < EHALF - count0, count0 + ip, E + ip)
    packed_p = (N + (ip % 240)) | (((ip % 128) * (HALF_N // 128)) << 16)
    pos = jnp.concatenate([pos_e, pos_p]).reshape(NTILES, NCHUNK, CH)
    packed = jnp.concatenate([packed_e, packed_p]).reshape(NTILES, NCHUNK, CH)
    zeros_i32 = jnp.zeros((2688,), jnp.int32)
    parts = _sc_part(pos, packed, zeros_i32)
    sflat, dflat = _tc_merge(parts[:EDGE_T].reshape(EDGE_T // 128, 128),
                             parts[EDGE_T:].reshape(EDGE_T // 128, 128))
    srcr2 = sflat.reshape(NTILES, NCHUNK, CH)
    dstr2 = dflat.reshape(NTILES, NCHUNK, CH)

    zeros_rows = jnp.zeros((64, H), jnp.float32)
    zeros_deg = jnp.zeros((RPS,), jnp.float32)
    ones_row = jnp.ones((CH,), jnp.float32)

    batch_pad = jnp.concatenate(
        [batch.astype(jnp.int32), jnp.full((NP - N,), G, jnp.int32)])
    ohng = (batch_pad[:, None] == jnp.arange(G, dtype=jnp.int32)[None, :]
            ).astype(jnp.float32)
    ohgn = ohng.T
    padmask = (jnp.arange(NP, dtype=jnp.int32) < N).astype(jnp.float32).reshape(RB, 128)
    ii = jnp.arange(128, dtype=jnp.int32)
    ut = (ii[:, None] <= ii[None, :]).astype(jnp.float32)
    rr = jnp.arange(RB, dtype=jnp.int32)
    slt = (rr[None, :] < rr[:, None]).astype(jnp.float32)
    gg = jnp.arange(G, dtype=jnp.int32)
    sut = (gg[:, None] < gg[None, :]).astype(jnp.float32)

    deg2 = _sc_deg(srcrd, ones_row, zeros_deg).reshape(2, RB, 128)
    x = _tc_enc(deg2, W_enc, b_enc, padmask)
    valid = padmask
    layer = 0
    for blk in range(NUM_BLOCKS):
        for _ in range(NUM_LAYERS):
            agg = _sc_agg(x, srcr2, dstr2, zeros_rows)
            x = _tc_layer(x, agg, W1[layer], b1[layer], W2[layer], b2[layer], valid)
            layer += 1
        if blk < NUM_BLOCKS - 1:
            w_blk = w_pool[blk].reshape(1, 1)
            x, valid = _tc_pool(x, w_blk, valid, ohng, ohgn, ut, slt, sut)
    return _tc_final(x, ohgn, Wd1, bd1, Wd2, bd2)
